# baseline (speedup 1.0000x reference)
import functools

import jax
import jax.numpy as jnp
from jax import lax
from jax.experimental import pallas as pl
from jax.experimental.pallas import tpu as pltpu
from jax.experimental.pallas import tpu_sc as plsc

B = 16384
D = 64
N_USERS = 1000000
N_ANIME = 100000
U_SHIFT = 20
A_SHIFT = 17
U_MAIN = N_USERS - N_USERS % 128
A_MAIN = N_ANIME - N_ANIME % 128
U_TAIL = N_USERS % 128
A_TAIL = N_ANIME % 128
NC = 2
NS = 16
NW = NC * NS
BPW = B // NW
CHUNK = 128
N_CHUNKS = BPW // CHUNK
LANES = 16
GROUPS = CHUNK // LANES
D_UNROLL = 4


def _flatten_body(n_main, n_tail, shift, in_ref, tail_ref, out_ref, sem):
    copies = [
        pltpu.make_async_copy(in_ref.at[d, pl.ds(0, n_main)],
                              out_ref.at[pl.ds(d << shift, n_main)], sem)
        for d in range(D)
    ]
    copies.append(
        pltpu.make_async_copy(tail_ref,
                              out_ref.at[pl.ds(D << shift, D * n_tail)],
                              sem))
    for c in copies:
        c.start()
    for c in copies:
        c.wait()


def _make_flatten(n, shift):
    n_main = n - n % 128
    n_tail = n % 128
    return pl.pallas_call(
        functools.partial(_flatten_body, n_main, n_tail, shift),
        in_specs=[pl.BlockSpec(memory_space=pl.ANY),
                  pl.BlockSpec(memory_space=pl.ANY)],
        out_specs=pl.BlockSpec(memory_space=pl.ANY),
        out_shape=jax.ShapeDtypeStruct(((D << shift) + D * n_tail,),
                                       jnp.float32),
        scratch_shapes=[pltpu.SemaphoreType.DMA],
    )


def _mf_body(user_hbm, anime_hbm, uf_hbm, af_hbm, out_hbm,
             uidx, aidx, uwidx, awidx, uvals, avals, outv, sems):
    wid = lax.axis_index("s") * NC + lax.axis_index("c")
    base = pl.multiple_of(wid * BPW, BPW)

    for k in range(N_CHUNKS):
        pltpu.sync_copy(user_hbm.at[pl.ds(base + k * CHUNK, CHUNK)],
                        uidx.at[k])
        pltpu.sync_copy(anime_hbm.at[pl.ds(base + k * CHUNK, CHUNK)],
                        aidx.at[k])

    def build_indices(k):
        buf = k % 2

        def per_d(d, carry):
            du = pl.multiple_of(d * (1 << U_SHIFT), 1 << U_SHIFT)
            da = pl.multiple_of(d * (1 << A_SHIFT), 1 << A_SHIFT)
            for g in range(GROUPS):
                sl = pl.ds(g * LANES, LANES)
                ru = uidx[k, sl]
                ra = aidx[k, sl]
                uwidx[buf, d, sl] = jnp.where(
                    ru < U_MAIN, ru + du,
                    ru + ((D << U_SHIFT) + d * U_TAIL - U_MAIN))
                awidx[buf, d, sl] = jnp.where(
                    ra < A_MAIN, ra + da,
                    ra + ((D << A_SHIFT) + d * A_TAIL - A_MAIN))
            return carry
        lax.fori_loop(0, D, per_d, 0)

    def fire(k):
        buf = k % 2

        def per_d(d, carry):
            pltpu.async_copy(uf_hbm.at[uwidx.at[buf, d]], uvals.at[buf, d],
                             sems.at[buf, 0])
            pltpu.async_copy(af_hbm.at[awidx.at[buf, d]], avals.at[buf, d],
                             sems.at[buf, 1])
            return carry
        lax.fori_loop(0, D, per_d, 0)

    def drain(k):
        buf = k % 2

        def per_d(d, carry):
            pltpu.make_async_copy(uf_hbm.at[uwidx.at[buf, d]],
                                  uvals.at[buf, d], sems.at[buf, 0]).wait()
            pltpu.make_async_copy(af_hbm.at[awidx.at[buf, d]],
                                  avals.at[buf, d], sems.at[buf, 1]).wait()
            return carry
        lax.fori_loop(0, D, per_d, 0)

    build_indices(0)
    fire(0)

    for k in range(N_CHUNKS):
        buf = k % 2
        if k + 1 < N_CHUNKS:
            build_indices(k + 1)
        drain(k)
        if k + 1 < N_CHUNKS:
            fire(k + 1)

        def group(g, carry, buf=buf):
            gbase = pl.multiple_of(g * LANES, LANES)
            sl = pl.ds(gbase, LANES)

            def per_d(d, acc, sl=sl):
                dd = pl.multiple_of(d * D_UNROLL, D_UNROLL)
                for i in range(D_UNROLL):
                    acc = acc + uvals[buf, dd + i, sl] * avals[buf, dd + i, sl]
                return acc

            acc = lax.fori_loop(0, D // D_UNROLL, per_d,
                                jnp.zeros((LANES,), jnp.float32))
            outv[pl.ds(pl.multiple_of(k * CHUNK, CHUNK) + gbase, LANES)] = acc
            return carry

        lax.fori_loop(0, GROUPS, group, 0)

    pltpu.sync_copy(outv, out_hbm.at[pl.ds(base, BPW)])


_mf_kernel = functools.partial(
    pl.kernel,
    out_type=jax.ShapeDtypeStruct((B,), jnp.float32),
    mesh=plsc.VectorSubcoreMesh(core_axis_name="c", subcore_axis_name="s"),
    scratch_types=[
        pltpu.VMEM((N_CHUNKS, CHUNK), jnp.int32),
        pltpu.VMEM((N_CHUNKS, CHUNK), jnp.int32),
        pltpu.VMEM((2, D, CHUNK), jnp.int32),
        pltpu.VMEM((2, D, CHUNK), jnp.int32),
        pltpu.VMEM((2, D, CHUNK), jnp.float32),
        pltpu.VMEM((2, D, CHUNK), jnp.float32),
        pltpu.VMEM((BPW,), jnp.float32),
        pltpu.SemaphoreType.DMA((2, 2)),
    ],
    compiler_params=pltpu.CompilerParams(use_tc_tiling_on_sc=False),
)(_mf_body)


def kernel(user, anime, user_factors, anime_factors):
    uf_t = jnp.swapaxes(user_factors, 0, 1)
    af_t = jnp.swapaxes(anime_factors, 0, 1)
    u_tail = lax.slice(uf_t, (0, U_MAIN), (D, N_USERS)).reshape(D * U_TAIL)
    a_tail = lax.slice(af_t, (0, A_MAIN), (D, N_ANIME)).reshape(D * A_TAIL)
    ufl = _make_flatten(N_USERS, U_SHIFT)(uf_t, u_tail)
    afl = _make_flatten(N_ANIME, A_SHIFT)(af_t, a_tail)
    return _mf_kernel(user.astype(jnp.int32), anime.astype(jnp.int32),
                      ufl, afl)


# speedup vs baseline: 6.2219x; 6.2219x over previous
import functools

import jax
import jax.numpy as jnp
from jax import lax
from jax.experimental import pallas as pl
from jax.experimental.pallas import tpu as pltpu
from jax.experimental.pallas import tpu_sc as plsc

B = 16384
D = 64
N_USERS = 1000000
N_ANIME = 100000
NC = 2
NS = 16
NW = NC * NS
BPW = B // NW
CHUNK = 128
N_CHUNKS = BPW // CHUNK
W = 512
DB = 512


def _pair_body(in_ref, out_ref):
    x = in_ref[...]
    y = jnp.swapaxes(x, 0, 1)
    out_ref[...] = jnp.concatenate([y[:W // 2], y[W // 2:]], axis=1)


def _make_pair(n):
    grid = (n + W - 1) // W
    return pl.pallas_call(
        _pair_body,
        grid=(grid,),
        in_specs=[pl.BlockSpec((D, W), lambda i: (0, i))],
        out_specs=pl.BlockSpec((W // 2, 128), lambda i: (i, 0)),
        out_shape=jax.ShapeDtypeStruct((grid * (W // 2), 128), jnp.float32),
    )


def _rho(v):
    return (jnp.bitwise_and(v, -512)
            + lax.shift_left(jnp.bitwise_and(v, 255), 1)
            + jnp.bitwise_and(lax.shift_right_logical(v, 8), 1))


def _gather_body(user_hbm, anime_hbm, uf_hbm, af_hbm, outu_hbm, outa_hbm,
                 uidx, aidx, urows, arows, sems):
    wid = lax.axis_index("s") * NC + lax.axis_index("c")
    base = pl.multiple_of(wid * BPW, BPW)

    for q in range(N_CHUNKS):
        pltpu.sync_copy(user_hbm.at[pl.ds(base + q * CHUNK, CHUNK)],
                        uidx.at[q])
        pltpu.sync_copy(anime_hbm.at[pl.ds(base + q * CHUNK, CHUNK)],
                        aidx.at[q])
        for j in range(CHUNK // 16):
            sl = pl.ds(j * 16, 16)
            uidx[q, sl] = _rho(uidx[q, sl])
            aidx[q, sl] = _rho(aidx[q, sl])

    def fire(q):
        buf = q % 2
        pltpu.async_copy(uf_hbm.at[uidx.at[q]], urows.at[buf],
                         sems.at[buf, 0])
        pltpu.async_copy(af_hbm.at[aidx.at[q]], arows.at[buf],
                         sems.at[buf, 1])

    def drain(q):
        buf = q % 2
        pltpu.make_async_copy(uf_hbm.at[uidx.at[q]], urows.at[buf],
                              sems.at[buf, 0]).wait()
        pltpu.make_async_copy(af_hbm.at[aidx.at[q]], arows.at[buf],
                              sems.at[buf, 1]).wait()

    fire(0)
    for q in range(N_CHUNKS):
        buf = q % 2
        if q + 1 < N_CHUNKS:
            fire(q + 1)
        drain(q)
        sl = pl.ds(base + q * CHUNK, CHUNK)
        pltpu.sync_copy(urows.at[buf], outu_hbm.at[sl])
        pltpu.sync_copy(arows.at[buf], outa_hbm.at[sl])


_gather_kernel = functools.partial(
    pl.kernel,
    out_type=(jax.ShapeDtypeStruct((B, D), jnp.float32),
              jax.ShapeDtypeStruct((B, D), jnp.float32)),
    mesh=plsc.VectorSubcoreMesh(core_axis_name="c", subcore_axis_name="s"),
    scratch_types=[
        pltpu.VMEM((N_CHUNKS, CHUNK), jnp.int32),
        pltpu.VMEM((N_CHUNKS, CHUNK), jnp.int32),
        pltpu.VMEM((2, CHUNK, D), jnp.float32),
        pltpu.VMEM((2, CHUNK, D), jnp.float32),
        pltpu.SemaphoreType.DMA((2, 2)),
    ],
    compiler_params=pltpu.CompilerParams(use_tc_tiling_on_sc=False),
)(_gather_body)


def _dot_body(u_ref, a_ref, s0_ref, s1_ref):
    p = u_ref[...] * a_ref[...]
    s0_ref[...] = jnp.sum(p[:, :D], axis=1)
    s1_ref[...] = jnp.sum(p[:, D:], axis=1)


_dot_kernel = pl.pallas_call(
    _dot_body,
    grid=(B // 2 // DB,),
    in_specs=[pl.BlockSpec((DB, 128), lambda i: (i, 0)),
              pl.BlockSpec((DB, 128), lambda i: (i, 0))],
    out_specs=(pl.BlockSpec((DB,), lambda i: (i,)),
               pl.BlockSpec((DB,), lambda i: (i,))),
    out_shape=(jax.ShapeDtypeStruct((B // 2,), jnp.float32),
               jax.ShapeDtypeStruct((B // 2,), jnp.float32)),
)


def kernel(user, anime, user_factors, anime_factors):
    ufp = _make_pair(N_USERS)(jnp.swapaxes(user_factors, 0, 1))
    afp = _make_pair(N_ANIME)(jnp.swapaxes(anime_factors, 0, 1))
    g_u, g_a = _gather_kernel(user.astype(jnp.int32),
                              anime.astype(jnp.int32),
                              ufp.reshape(ufp.shape[0] * 2, D),
                              afp.reshape(afp.shape[0] * 2, D))
    s0, s1 = _dot_kernel(g_u.reshape(B // 2, 128), g_a.reshape(B // 2, 128))
    return jnp.stack([s0, s1], axis=1).reshape(B)


# speedup vs baseline: 12.5180x; 2.0119x over previous
import functools

import jax
import jax.numpy as jnp
from jax import lax
from jax.experimental import pallas as pl
from jax.experimental.pallas import tpu as pltpu
from jax.experimental.pallas import tpu_sc as plsc

B = 16384
D = 64
N_USERS = 1000000
N_ANIME = 100000
NC = 2
NS = 16
NW = NC * NS
BPW = B // NW
CHUNK = 128
N_CHUNKS = BPW // CHUNK
W = 512
DB = 512


def _pair_body(in_ref, out_ref):
    x = in_ref[...]
    y = jnp.swapaxes(x, 0, 1)
    out_ref[...] = jnp.concatenate([y[:W // 2], y[W // 2:]], axis=1)


def _make_pair(n):
    grid = (n + W - 1) // W
    return pl.pallas_call(
        _pair_body,
        grid=(grid,),
        in_specs=[pl.BlockSpec((D, W), lambda i: (0, i))],
        out_specs=pl.BlockSpec((W // 2, 128), lambda i: (i, 0)),
        out_shape=jax.ShapeDtypeStruct((grid * (W // 2), 128), jnp.float32),
    )


def _rho(v):
    return (jnp.bitwise_and(v, -512)
            + lax.shift_left(jnp.bitwise_and(v, 255), 1)
            + jnp.bitwise_and(lax.shift_right_logical(v, 8), 1))


def _gather_body(user_hbm, anime_hbm, uf_hbm, af_hbm, outu_hbm, outa_hbm,
                 uidx, aidx, urows, arows, sems):
    wid = lax.axis_index("s") * NC + lax.axis_index("c")
    base = pl.multiple_of(wid * BPW, BPW)

    for q in range(N_CHUNKS):
        pltpu.sync_copy(user_hbm.at[pl.ds(base + q * CHUNK, CHUNK)],
                        uidx.at[q])
        pltpu.sync_copy(anime_hbm.at[pl.ds(base + q * CHUNK, CHUNK)],
                        aidx.at[q])

    def fire(q):
        buf = q % 2
        pltpu.async_copy(uf_hbm.at[uidx.at[q]], urows.at[buf],
                         sems.at[buf, 0])
        pltpu.async_copy(af_hbm.at[aidx.at[q]], arows.at[buf],
                         sems.at[buf, 1])

    def drain(q):
        buf = q % 2
        pltpu.make_async_copy(uf_hbm.at[uidx.at[q]], urows.at[buf],
                              sems.at[buf, 0]).wait()
        pltpu.make_async_copy(af_hbm.at[aidx.at[q]], arows.at[buf],
                              sems.at[buf, 1]).wait()

    fire(0)
    for q in range(N_CHUNKS):
        buf = q % 2
        if q + 1 < N_CHUNKS:
            fire(q + 1)
        drain(q)
        sl = pl.ds(base + q * CHUNK, CHUNK)
        pltpu.sync_copy(urows.at[buf], outu_hbm.at[sl])
        pltpu.sync_copy(arows.at[buf], outa_hbm.at[sl])


_gather_kernel = functools.partial(
    pl.kernel,
    out_type=(jax.ShapeDtypeStruct((B, D), jnp.float32),
              jax.ShapeDtypeStruct((B, D), jnp.float32)),
    mesh=plsc.VectorSubcoreMesh(core_axis_name="c", subcore_axis_name="s"),
    scratch_types=[
        pltpu.VMEM((N_CHUNKS, CHUNK), jnp.int32),
        pltpu.VMEM((N_CHUNKS, CHUNK), jnp.int32),
        pltpu.VMEM((2, CHUNK, D), jnp.float32),
        pltpu.VMEM((2, CHUNK, D), jnp.float32),
        pltpu.SemaphoreType.DMA((2, 2)),
    ],
    compiler_params=pltpu.CompilerParams(use_tc_tiling_on_sc=False),
)(_gather_body)


def _dot_body(u_ref, a_ref, s0_ref, s1_ref):
    p = u_ref[...] * a_ref[...]
    s0_ref[...] = jnp.sum(p[:, :D], axis=1)
    s1_ref[...] = jnp.sum(p[:, D:], axis=1)


_dot_kernel = pl.pallas_call(
    _dot_body,
    grid=(B // 2 // DB,),
    in_specs=[pl.BlockSpec((DB, 128), lambda i: (i, 0)),
              pl.BlockSpec((DB, 128), lambda i: (i, 0))],
    out_specs=(pl.BlockSpec((DB,), lambda i: (i,)),
               pl.BlockSpec((DB,), lambda i: (i,))),
    out_shape=(jax.ShapeDtypeStruct((B // 2,), jnp.float32),
               jax.ShapeDtypeStruct((B // 2,), jnp.float32)),
)


def kernel(user, anime, user_factors, anime_factors):
    g_u, g_a = _gather_kernel(user.astype(jnp.int32),
                              anime.astype(jnp.int32),
                              user_factors, anime_factors)
    s0, s1 = _dot_kernel(g_u.reshape(B // 2, 128), g_a.reshape(B // 2, 128))
    return jnp.stack([s0, s1], axis=1).reshape(B)
